# baseline (speedup 1.0000x reference)
import functools

import jax
import jax.numpy as jnp
from jax import lax
from jax.experimental import pallas as pl
from jax.experimental.pallas import tpu as pltpu
from jax.experimental.pallas import tpu_sc as plsc

N = 10000
E = 320000
D = 128
R = 7
NB = 4

NC = 2
NS = 16
NT = NC * NS
EPT = E // NT
NV = EPT // 16
CHUNK = 128
CAPB = -(-EPT // CHUNK)
CAP = CAPB * CHUNK
LBUF = CAP + CHUNK
NPAD = 10240
HALF = NPAD // 2
RPT = HALF // NS
NULL = N

_mesh = plsc.VectorSubcoreMesh(
    core_axis_name="c", subcore_axis_name="s", num_cores=NC, num_subcores=NS)


@functools.partial(
    pl.kernel,
    out_type=(
        jax.ShapeDtypeStruct((2 * NT * R * CAP,), jnp.int32),
        jax.ShapeDtypeStruct((2 * NT * R * CAP,), jnp.int32),
        jax.ShapeDtypeStruct((2 * NT * 16,), jnp.int32),
    ),
    mesh=_mesh,
    scratch_types=[
        pltpu.VMEM((EPT,), jnp.int32),
        pltpu.VMEM((EPT,), jnp.int32),
        pltpu.VMEM((EPT,), jnp.int32),
        pltpu.VMEM((LBUF,), jnp.int32),
        pltpu.VMEM((LBUF,), jnp.int32),
        pltpu.VMEM((LBUF,), jnp.int32),
        pltpu.VMEM((LBUF,), jnp.int32),
        pltpu.VMEM((16,), jnp.int32),
    ],
    compiler_params=pltpu.CompilerParams(needs_layout_passes=False),
)
def _bucket_kernel(src_hbm, dst_hbm, et_hbm, srcl_hbm, dstl_hbm, cnts_hbm,
                   esrc, edst, eet, ls0, ld0, ls1, ld1, cbuf):
    c = lax.axis_index("c")
    s = lax.axis_index("s")
    wid = s * NC + c
    base = wid * EPT
    pltpu.sync_copy(src_hbm.at[pl.ds(base, EPT)], esrc)
    pltpu.sync_copy(dst_hbm.at[pl.ds(base, EPT)], edst)
    pltpu.sync_copy(et_hbm.at[pl.ds(base, EPT)], eet)
    lanes = lax.iota(jnp.int32, 16)
    nullv = jnp.full((16,), NULL, jnp.int32)
    zerov = jnp.zeros((16,), jnp.int32)
    tvec0 = jnp.zeros((16,), jnp.int32)
    tvec1 = jnp.zeros((16,), jnp.int32)
    for r in range(R):
        def body(i, carry, r=r):
            cnt0, cnt1 = carry
            off = i * 16
            ev = eet[pl.ds(off, 16)]
            sv = esrc[pl.ds(off, 16)]
            dv = edst[pl.ds(off, 16)]
            m = ev == r
            hi = dv >= HALF
            m0 = m & (~hi)
            m1 = m & hi
            mi0 = m0.astype(jnp.int32)
            incl0 = plsc.cumsum(mi0)
            pos0 = cnt0 + incl0 - mi0
            plsc.store_scatter(ls0, [pos0], sv, mask=m0)
            plsc.store_scatter(ld0, [pos0], dv, mask=m0)
            mi1 = m1.astype(jnp.int32)
            incl1 = plsc.cumsum(mi1)
            pos1 = cnt1 + incl1 - mi1
            plsc.store_scatter(ls1, [pos1], sv, mask=m1)
            plsc.store_scatter(ld1, [pos1], dv - HALF, mask=m1)
            return (cnt0 + jnp.max(incl0), cnt1 + jnp.max(incl1))
        cnt0, cnt1 = lax.fori_loop(0, NV, body, (jnp.int32(0), jnp.int32(0)))
        for k in range(CHUNK // 16):
            ls0[pl.ds(cnt0 + k * 16, 16)] = nullv
            ld0[pl.ds(cnt0 + k * 16, 16)] = zerov
            ls1[pl.ds(cnt1 + k * 16, 16)] = nullv
            ld1[pl.ds(cnt1 + k * 16, 16)] = zerov
        trips0 = (cnt0 + (CHUNK - 1)) // CHUNK
        trips1 = (cnt1 + (CHUNK - 1)) // CHUNK
        tvec0 = jnp.where(lanes == r, trips0, tvec0)
        tvec1 = jnp.where(lanes == r, trips1, tvec1)
        lb0 = (wid * R + r) * CAP
        lb1 = ((NT + wid) * R + r) * CAP
        pltpu.sync_copy(ls0.at[pl.ds(0, CAP)], srcl_hbm.at[pl.ds(lb0, CAP)])
        pltpu.sync_copy(ld0.at[pl.ds(0, CAP)], dstl_hbm.at[pl.ds(lb0, CAP)])
        pltpu.sync_copy(ls1.at[pl.ds(0, CAP)], srcl_hbm.at[pl.ds(lb1, CAP)])
        pltpu.sync_copy(ld1.at[pl.ds(0, CAP)], dstl_hbm.at[pl.ds(lb1, CAP)])
    cbuf[...] = tvec0
    pltpu.sync_copy(cbuf, cnts_hbm.at[pl.ds(wid * 16, 16)])
    cbuf[...] = tvec1
    pltpu.sync_copy(cbuf, cnts_hbm.at[pl.ds((NT + wid) * 16, 16)])


@functools.partial(
    pl.kernel,
    out_type=jax.ShapeDtypeStruct((R, NPAD, D), jnp.float32),
    mesh=_mesh,
    scratch_types=[
        pltpu.VMEM((CAPB, CHUNK), jnp.int32),
        pltpu.VMEM((CAPB, CHUNK), jnp.int32),
        pltpu.VMEM((CHUNK, D), jnp.float32),
        pltpu.VMEM((CHUNK, D), jnp.float32),
        pltpu.VMEM((CHUNK, D), jnp.float32),
        pltpu.VMEM((16,), jnp.int32),
        pltpu.VMEM((16,), jnp.int32),
        pltpu.SemaphoreType.DMA,
        pltpu.SemaphoreType.DMA,
        pltpu.SemaphoreType.DMA,
        pltpu.VMEM_SHARED((HALF, D), jnp.float32),
    ],
    compiler_params=pltpu.CompilerParams(needs_layout_passes=False),
)
def _scatter_kernel(x_hbm, srcl_hbm, dstl_hbm, cnts_hbm, zs_hbm, S_hbm,
                    sidx, didx, gb0, gb1, gb2, cbufa, cbufb,
                    sem0, sem1, sem2, S_sh):
    NBUF = 3
    gbufs = (gb0, gb1, gb2)
    sems = (sem0, sem1, sem2)
    c = lax.axis_index("c")
    s = lax.axis_index("s")
    pltpu.sync_copy(cnts_hbm.at[pl.ds((c * NT + s) * 16, 16)], cbufa)
    pltpu.sync_copy(cnts_hbm.at[pl.ds((c * NT + s + NS) * 16, 16)], cbufb)
    lanes = lax.iota(jnp.int32, 16)
    for r in range(R):
        pltpu.sync_copy(zs_hbm, S_sh.at[pl.ds(s * RPT, CHUNK)])
        pltpu.sync_copy(zs_hbm, S_sh.at[pl.ds(s * RPT + CHUNK, CHUNK)])
        pltpu.sync_copy(zs_hbm.at[pl.ds(0, RPT - 2 * CHUNK)],
                        S_sh.at[pl.ds(s * RPT + 2 * CHUNK, RPT - 2 * CHUNK)])
        plsc.subcore_barrier()
        for k in range(2):
            b = s + k * NS
            cvec = (cbufa if k == 0 else cbufb)[...]
            trips = jnp.max(jnp.where(lanes == r, cvec, 0))
            nstage = (trips + 7) // 8

            def stage_body(bk, carry, b=b):
                pltpu.sync_copy(srcl_hbm.at[c, b, r, pl.ds(bk * 8, 8)],
                                sidx.at[pl.ds(bk * 8, 8)])
                pltpu.sync_copy(dstl_hbm.at[c, b, r, pl.ds(bk * 8, 8)],
                                didx.at[pl.ds(bk * 8, 8)])
                return carry
            lax.fori_loop(0, nstage, stage_body, jnp.int32(0))

            for t in range(NBUF):
                @pl.when(t < trips)
                def _(t=t):
                    pltpu.async_copy(x_hbm.at[sidx.at[t]], gbufs[t], sems[t])

            def ring_body(g, carry, trips=trips):
                for t in range(NBUF):
                    j = g * NBUF + t

                    @pl.when(j < trips)
                    def _(j=j, t=t):
                        pltpu.make_async_copy(
                            x_hbm.at[sidx.at[j]], gbufs[t], sems[t]).wait()
                        pltpu.sync_copy(gbufs[t], S_sh.at[didx.at[j]],
                                        add=True)

                    @pl.when(j + NBUF < trips)
                    def _(j=j, t=t):
                        pltpu.async_copy(
                            x_hbm.at[sidx.at[j + NBUF]], gbufs[t], sems[t])
                return carry
            lax.fori_loop(0, (trips + NBUF - 1) // NBUF, ring_body,
                          jnp.int32(0))
        plsc.subcore_barrier()
        pltpu.sync_copy(S_sh.at[pl.ds(s * RPT, RPT)],
                        S_hbm.at[r, pl.ds(c * HALF + s * RPT, RPT)])


BN = 1024
NBLK = NPAD // BN


def _dense_body(x_ref, S_ref, deg_ref, bases_ref, coeffs_ref, wsl_ref,
                bsl_ref, gamma_ref, beta_ref, out_ref):
    i = pl.program_id(0)
    x = x_ref[...]
    h = jnp.dot(x, wsl_ref[...], preferred_element_type=jnp.float32)
    h = h + bsl_ref[...]
    bases = bases_ref[...]
    cp = coeffs_ref[...]
    for r in range(R):
        W_r = jnp.zeros((D, D), jnp.float32)
        for b in range(NB):
            W_r = W_r + cp[r, b] * bases[b]
        inv = 1.0 / jnp.maximum(deg_ref[r], 1.0)
        h = h + jnp.dot(S_ref[r], W_r, preferred_element_type=jnp.float32) * inv
    h = h * (1.0 / (1.0 + jnp.exp(-h)))
    mu = jnp.mean(h, axis=-1, keepdims=True)
    var = jnp.mean((h - mu) ** 2, axis=-1, keepdims=True)
    h = (h - mu) * lax.rsqrt(var + 1e-5) * gamma_ref[...] + beta_ref[...]
    rowid = i * BN + lax.broadcasted_iota(jnp.int32, (BN, 1), 0)
    out_ref[...] = jnp.where(rowid < N, h, 0.0)


_dense = pl.pallas_call(
    _dense_body,
    grid=(NBLK,),
    in_specs=[
        pl.BlockSpec((BN, D), lambda i: (i, 0)),
        pl.BlockSpec((R, BN, D), lambda i: (0, i, 0)),
        pl.BlockSpec((R, BN, 1), lambda i: (0, i, 0)),
        pl.BlockSpec((NB, D, D), lambda i: (0, 0, 0)),
        pl.BlockSpec((8, 8), lambda i: (0, 0)),
        pl.BlockSpec((D, D), lambda i: (0, 0)),
        pl.BlockSpec((1, D), lambda i: (0, 0)),
        pl.BlockSpec((1, D), lambda i: (0, 0)),
        pl.BlockSpec((1, D), lambda i: (0, 0)),
    ],
    out_specs=pl.BlockSpec((BN, D), lambda i: (i, 0)),
    out_shape=jax.ShapeDtypeStruct((NPAD, D), jnp.float32),
)


def kernel(x, edge_index, edge_type,
           bases_0, coeffs_0, Wsl_0, bsl_0, gamma_0, beta_0,
           bases_1, coeffs_1, Wsl_1, bsl_1, gamma_1, beta_1,
           bases_2, coeffs_2, Wsl_2, bsl_2, gamma_2, beta_2):
    src = edge_index[0].astype(jnp.int32)
    dst = edge_index[1].astype(jnp.int32)
    et = edge_type.astype(jnp.int32)
    x_pad = jnp.zeros((NPAD, D), jnp.float32).at[:N].set(x)
    xdeg = jnp.zeros((NPAD, D), jnp.float32).at[:N].set(1.0)

    srcl, dstl, cnts = _bucket_kernel(src, dst, et)
    srcl5 = srcl.reshape(2, NT, R, CAPB, CHUNK)
    dstl5 = dstl.reshape(2, NT, R, CAPB, CHUNK)

    zs = jnp.zeros((CHUNK, D), jnp.float32)

    Sdeg = _scatter_kernel(xdeg, srcl5, dstl5, cnts, zs)
    deg1 = Sdeg[:, :, 0:1]

    params = [
        (bases_0, coeffs_0, Wsl_0, bsl_0, gamma_0, beta_0),
        (bases_1, coeffs_1, Wsl_1, bsl_1, gamma_1, beta_1),
        (bases_2, coeffs_2, Wsl_2, bsl_2, gamma_2, beta_2),
    ]
    h = x_pad
    for bases, coeffs, wsl, bsl, gamma, beta in params:
        S = _scatter_kernel(h, srcl5, dstl5, cnts, zs)
        cpad = jnp.zeros((8, 8), jnp.float32).at[:R, :NB].set(coeffs)
        h = _dense(h, S, deg1, bases, cpad, wsl,
                   bsl.reshape(1, D), gamma.reshape(1, D), beta.reshape(1, D))
    return h[:N]


# speedup vs baseline: 1.2629x; 1.2629x over previous
import functools

import jax
import jax.numpy as jnp
from jax import lax
from jax.experimental import pallas as pl
from jax.experimental.pallas import tpu as pltpu
from jax.experimental.pallas import tpu_sc as plsc

N = 10000
E = 320000
D = 128
R = 7
NB = 4

NC = 2
NS = 16
NT = NC * NS
EPT = E // NT
NV = EPT // 16
CHUNK = 128
CAPB = -(-EPT // CHUNK)
CAP = CAPB * CHUNK
LBUF = CAP + CHUNK
NPAD = 10240
HALF = NPAD // 2
HDIM = HALF + 8
RPT = HALF // NS
NULL = N

_mesh = plsc.VectorSubcoreMesh(
    core_axis_name="c", subcore_axis_name="s", num_cores=NC, num_subcores=NS)


@functools.partial(
    pl.kernel,
    out_type=(
        jax.ShapeDtypeStruct((2 * NT * R * CAP,), jnp.int32),
        jax.ShapeDtypeStruct((2 * NT * R * CAP,), jnp.int32),
        jax.ShapeDtypeStruct((2 * NT * 16,), jnp.int32),
    ),
    mesh=_mesh,
    scratch_types=[
        pltpu.VMEM((EPT,), jnp.int32),
        pltpu.VMEM((EPT,), jnp.int32),
        pltpu.VMEM((EPT,), jnp.int32),
        pltpu.VMEM((LBUF,), jnp.int32),
        pltpu.VMEM((LBUF,), jnp.int32),
        pltpu.VMEM((LBUF,), jnp.int32),
        pltpu.VMEM((LBUF,), jnp.int32),
        pltpu.VMEM((16,), jnp.int32),
    ],
    compiler_params=pltpu.CompilerParams(needs_layout_passes=False),
)
def _bucket_kernel(src_hbm, dst_hbm, et_hbm, srcl_hbm, dstl_hbm, cnts_hbm,
                   esrc, edst, eet, ls0, ld0, ls1, ld1, cbuf):
    c = lax.axis_index("c")
    s = lax.axis_index("s")
    wid = s * NC + c
    base = wid * EPT
    pltpu.sync_copy(src_hbm.at[pl.ds(base, EPT)], esrc)
    pltpu.sync_copy(dst_hbm.at[pl.ds(base, EPT)], edst)
    pltpu.sync_copy(et_hbm.at[pl.ds(base, EPT)], eet)
    lanes = lax.iota(jnp.int32, 16)
    nullv = jnp.full((16,), NULL, jnp.int32)
    wastev = jnp.full((16,), HALF, jnp.int32)
    tvec0 = jnp.zeros((16,), jnp.int32)
    tvec1 = jnp.zeros((16,), jnp.int32)
    for r in range(R):
        def body(i, carry, r=r):
            cnt0, cnt1 = carry
            off = i * 16
            ev = eet[pl.ds(off, 16)]
            sv = esrc[pl.ds(off, 16)]
            dv = edst[pl.ds(off, 16)]
            m = ev == r
            hi = dv >= HALF
            m0 = m & (~hi)
            m1 = m & hi
            mi0 = m0.astype(jnp.int32)
            incl0 = plsc.cumsum(mi0)
            pos0 = cnt0 + incl0 - mi0
            plsc.store_scatter(ls0, [pos0], sv, mask=m0)
            plsc.store_scatter(ld0, [pos0], dv, mask=m0)
            mi1 = m1.astype(jnp.int32)
            incl1 = plsc.cumsum(mi1)
            pos1 = cnt1 + incl1 - mi1
            plsc.store_scatter(ls1, [pos1], sv, mask=m1)
            plsc.store_scatter(ld1, [pos1], dv - HALF, mask=m1)
            return (cnt0 + jnp.max(incl0), cnt1 + jnp.max(incl1))
        cnt0, cnt1 = lax.fori_loop(0, NV, body, (jnp.int32(0), jnp.int32(0)))
        for k in range(CHUNK // 16):
            ls0[pl.ds(cnt0 + k * 16, 16)] = nullv
            ld0[pl.ds(cnt0 + k * 16, 16)] = wastev
            ls1[pl.ds(cnt1 + k * 16, 16)] = nullv
            ld1[pl.ds(cnt1 + k * 16, 16)] = wastev
        trips0 = (cnt0 + (CHUNK - 1)) // CHUNK
        trips1 = (cnt1 + (CHUNK - 1)) // CHUNK
        tvec0 = jnp.where(lanes == r, trips0, tvec0)
        tvec1 = jnp.where(lanes == r, trips1, tvec1)
        lb0 = (wid * R + r) * CAP
        lb1 = ((NT + wid) * R + r) * CAP
        pltpu.sync_copy(ls0.at[pl.ds(0, CAP)], srcl_hbm.at[pl.ds(lb0, CAP)])
        pltpu.sync_copy(ld0.at[pl.ds(0, CAP)], dstl_hbm.at[pl.ds(lb0, CAP)])
        pltpu.sync_copy(ls1.at[pl.ds(0, CAP)], srcl_hbm.at[pl.ds(lb1, CAP)])
        pltpu.sync_copy(ld1.at[pl.ds(0, CAP)], dstl_hbm.at[pl.ds(lb1, CAP)])
    cbuf[...] = tvec0
    pltpu.sync_copy(cbuf, cnts_hbm.at[pl.ds(wid * 16, 16)])
    cbuf[...] = tvec1
    pltpu.sync_copy(cbuf, cnts_hbm.at[pl.ds((NT + wid) * 16, 16)])


@functools.partial(
    pl.kernel,
    out_type=jax.ShapeDtypeStruct((R, NPAD, D), jnp.float32),
    mesh=_mesh,
    scratch_types=[
        pltpu.VMEM((CAPB, CHUNK), jnp.int32),
        pltpu.VMEM((CAPB, CHUNK), jnp.int32),
        pltpu.VMEM((CHUNK, D), jnp.float32),
        pltpu.VMEM((CHUNK, D), jnp.float32),
        pltpu.VMEM((CHUNK, D), jnp.float32),
        pltpu.VMEM((16,), jnp.int32),
        pltpu.VMEM((16,), jnp.int32),
        pltpu.SemaphoreType.DMA,
        pltpu.SemaphoreType.DMA,
        pltpu.SemaphoreType.DMA,
        pltpu.VMEM_SHARED((HDIM, D), jnp.float32),
    ],
    compiler_params=pltpu.CompilerParams(needs_layout_passes=False),
)
def _scatter_kernel(x_hbm, srcl_hbm, dstl_hbm, cnts_hbm, zs_hbm, S_hbm,
                    sidx, didx, gb0, gb1, gb2, cbufa, cbufb,
                    sem0, sem1, sem2, S_sh):
    NBUF = 3
    gbufs = (gb0, gb1, gb2)
    sems = (sem0, sem1, sem2)
    c = lax.axis_index("c")
    s = lax.axis_index("s")
    pltpu.sync_copy(cnts_hbm.at[pl.ds((c * NT + s) * 16, 16)], cbufa)
    pltpu.sync_copy(cnts_hbm.at[pl.ds((c * NT + s + NS) * 16, 16)], cbufb)
    lanes = lax.iota(jnp.int32, 16)
    for r in range(R):
        pltpu.sync_copy(zs_hbm, S_sh.at[pl.ds(s * RPT, CHUNK)])
        pltpu.sync_copy(zs_hbm, S_sh.at[pl.ds(s * RPT + CHUNK, CHUNK)])
        pltpu.sync_copy(zs_hbm.at[pl.ds(0, RPT - 2 * CHUNK)],
                        S_sh.at[pl.ds(s * RPT + 2 * CHUNK, RPT - 2 * CHUNK)])
        plsc.subcore_barrier()
        for k in range(2):
            b = s + k * NS
            cvec = (cbufa if k == 0 else cbufb)[...]
            trips = jnp.max(jnp.where(lanes == r, cvec, 0))
            nstage = (trips + 7) // 8

            def stage_body(bk, carry, b=b):
                pltpu.sync_copy(srcl_hbm.at[c, b, r, pl.ds(bk * 8, 8)],
                                sidx.at[pl.ds(bk * 8, 8)])
                pltpu.sync_copy(dstl_hbm.at[c, b, r, pl.ds(bk * 8, 8)],
                                didx.at[pl.ds(bk * 8, 8)])
                return carry
            lax.fori_loop(0, nstage, stage_body, jnp.int32(0))

            for t in range(NBUF):
                @pl.when(t < trips)
                def _(t=t):
                    pltpu.async_copy(x_hbm.at[sidx.at[t]], gbufs[t], sems[t])

            def ring_body(g, carry, trips=trips):
                for t in range(NBUF):
                    j = g * NBUF + t

                    @pl.when(j < trips)
                    def _(j=j, t=t):
                        pltpu.make_async_copy(
                            x_hbm.at[sidx.at[j]], gbufs[t], sems[t]).wait()
                        pltpu.sync_copy(gbufs[t], S_sh.at[didx.at[j]],
                                        add=True)

                    @pl.when(j + NBUF < trips)
                    def _(j=j, t=t):
                        pltpu.async_copy(
                            x_hbm.at[sidx.at[j + NBUF]], gbufs[t], sems[t])
                return carry
            lax.fori_loop(0, (trips + NBUF - 1) // NBUF, ring_body,
                          jnp.int32(0))
        plsc.subcore_barrier()
        pltpu.sync_copy(S_sh.at[pl.ds(s * RPT, RPT)],
                        S_hbm.at[r, pl.ds(c * HALF + s * RPT, RPT)])


@functools.partial(
    pl.kernel,
    out_type=jax.ShapeDtypeStruct((R, NPAD, D), jnp.float32),
    mesh=_mesh,
    scratch_types=[
        pltpu.VMEM((CAPB, CHUNK), jnp.int32),
        pltpu.VMEM((CHUNK, D), jnp.float32),
        pltpu.VMEM((16,), jnp.int32),
        pltpu.VMEM((16,), jnp.int32),
        pltpu.VMEM_SHARED((HDIM, D), jnp.float32),
    ],
    compiler_params=pltpu.CompilerParams(needs_layout_passes=False),
)
def _deg_kernel(dstl_hbm, cnts_hbm, zs_hbm, ones_hbm, deg_hbm,
                didx, ones_b, cbufa, cbufb, deg_sh):
    c = lax.axis_index("c")
    s = lax.axis_index("s")
    pltpu.sync_copy(ones_hbm, ones_b)
    pltpu.sync_copy(cnts_hbm.at[pl.ds((c * NT + s) * 16, 16)], cbufa)
    pltpu.sync_copy(cnts_hbm.at[pl.ds((c * NT + s + NS) * 16, 16)], cbufb)
    lanes = lax.iota(jnp.int32, 16)
    for r in range(R):
        pltpu.sync_copy(zs_hbm, deg_sh.at[pl.ds(s * RPT, CHUNK)])
        pltpu.sync_copy(zs_hbm, deg_sh.at[pl.ds(s * RPT + CHUNK, CHUNK)])
        pltpu.sync_copy(zs_hbm.at[pl.ds(0, RPT - 2 * CHUNK)],
                        deg_sh.at[pl.ds(s * RPT + 2 * CHUNK, RPT - 2 * CHUNK)])
        plsc.subcore_barrier()
        for k in range(2):
            b = s + k * NS
            cvec = (cbufa if k == 0 else cbufb)[...]
            trips = jnp.max(jnp.where(lanes == r, cvec, 0))
            nstage = (trips + 7) // 8

            def stage_body(bk, carry, b=b):
                pltpu.sync_copy(dstl_hbm.at[c, b, r, pl.ds(bk * 8, 8)],
                                didx.at[pl.ds(bk * 8, 8)])
                return carry
            lax.fori_loop(0, nstage, stage_body, jnp.int32(0))

            def chunk_body(j, carry):
                pltpu.sync_copy(ones_b, deg_sh.at[didx.at[j]], add=True)
                return carry
            lax.fori_loop(0, trips, chunk_body, jnp.int32(0))
        plsc.subcore_barrier()
        pltpu.sync_copy(deg_sh.at[pl.ds(s * RPT, RPT)],
                        deg_hbm.at[r, pl.ds(c * HALF + s * RPT, RPT)])


BN = 1024
NBLK = NPAD // BN


def _dense_body(x_ref, S_ref, deg_ref, bases_ref, coeffs_ref, wsl_ref,
                bsl_ref, gamma_ref, beta_ref, out_ref):
    i = pl.program_id(0)
    x = x_ref[...]
    h = jnp.dot(x, wsl_ref[...], preferred_element_type=jnp.float32)
    h = h + bsl_ref[...]
    bases = bases_ref[...]
    cp = coeffs_ref[...]
    for r in range(R):
        W_r = jnp.zeros((D, D), jnp.float32)
        for b in range(NB):
            W_r = W_r + cp[r, b] * bases[b]
        inv = 1.0 / jnp.maximum(deg_ref[r], 1.0)
        h = h + jnp.dot(S_ref[r], W_r, preferred_element_type=jnp.float32) * inv
    h = h * (1.0 / (1.0 + jnp.exp(-h)))
    mu = jnp.mean(h, axis=-1, keepdims=True)
    var = jnp.mean((h - mu) ** 2, axis=-1, keepdims=True)
    h = (h - mu) * lax.rsqrt(var + 1e-5) * gamma_ref[...] + beta_ref[...]
    rowid = i * BN + lax.broadcasted_iota(jnp.int32, (BN, 1), 0)
    out_ref[...] = jnp.where(rowid < N, h, 0.0)


_dense = pl.pallas_call(
    _dense_body,
    grid=(NBLK,),
    in_specs=[
        pl.BlockSpec((BN, D), lambda i: (i, 0)),
        pl.BlockSpec((R, BN, D), lambda i: (0, i, 0)),
        pl.BlockSpec((R, BN, 1), lambda i: (0, i, 0)),
        pl.BlockSpec((NB, D, D), lambda i: (0, 0, 0)),
        pl.BlockSpec((8, 8), lambda i: (0, 0)),
        pl.BlockSpec((D, D), lambda i: (0, 0)),
        pl.BlockSpec((1, D), lambda i: (0, 0)),
        pl.BlockSpec((1, D), lambda i: (0, 0)),
        pl.BlockSpec((1, D), lambda i: (0, 0)),
    ],
    out_specs=pl.BlockSpec((BN, D), lambda i: (i, 0)),
    out_shape=jax.ShapeDtypeStruct((NPAD, D), jnp.float32),
)


def kernel(x, edge_index, edge_type,
           bases_0, coeffs_0, Wsl_0, bsl_0, gamma_0, beta_0,
           bases_1, coeffs_1, Wsl_1, bsl_1, gamma_1, beta_1,
           bases_2, coeffs_2, Wsl_2, bsl_2, gamma_2, beta_2):
    src = edge_index[0].astype(jnp.int32)
    dst = edge_index[1].astype(jnp.int32)
    et = edge_type.astype(jnp.int32)
    x_pad = jnp.zeros((NPAD, D), jnp.float32).at[:N].set(x)

    srcl, dstl, cnts = _bucket_kernel(src, dst, et)
    srcl5 = srcl.reshape(2, NT, R, CAPB, CHUNK)
    dstl5 = dstl.reshape(2, NT, R, CAPB, CHUNK)

    zs = jnp.zeros((CHUNK, D), jnp.float32)
    ones128 = jnp.ones((CHUNK, D), jnp.float32)

    deg = _deg_kernel(dstl5, cnts, zs, ones128)
    deg1 = deg[:, :, 0:1]

    params = [
        (bases_0, coeffs_0, Wsl_0, bsl_0, gamma_0, beta_0),
        (bases_1, coeffs_1, Wsl_1, bsl_1, gamma_1, beta_1),
        (bases_2, coeffs_2, Wsl_2, bsl_2, gamma_2, beta_2),
    ]
    h = x_pad
    for bases, coeffs, wsl, bsl, gamma, beta in params:
        S = _scatter_kernel(h, srcl5, dstl5, cnts, zs)
        cpad = jnp.zeros((8, 8), jnp.float32).at[:R, :NB].set(coeffs)
        h = _dense(h, S, deg1, bases, cpad, wsl,
                   bsl.reshape(1, D), gamma.reshape(1, D), beta.reshape(1, D))
    return h[:N]


# speedup vs baseline: 1.2780x; 1.0119x over previous
import functools

import jax
import jax.numpy as jnp
from jax import lax
from jax.experimental import pallas as pl
from jax.experimental.pallas import tpu as pltpu
from jax.experimental.pallas import tpu_sc as plsc

N = 10000
E = 320000
D = 128
R = 7
NB = 4

NC = 2
NS = 16
NT = NC * NS
EPT = E // NT
NV = EPT // 16
CHUNK = 128
CAPB = -(-EPT // CHUNK)
CAP = CAPB * CHUNK
LBUF = CAP + CHUNK
NPAD = 10240
HALF = NPAD // 2
HDIM = HALF + 8
RPT = HALF // NS
NULL = N // 2

_mesh = plsc.VectorSubcoreMesh(
    core_axis_name="c", subcore_axis_name="s", num_cores=NC, num_subcores=NS)


@functools.partial(
    pl.kernel,
    out_type=(
        jax.ShapeDtypeStruct((2 * NT * R * CAP,), jnp.int32),
        jax.ShapeDtypeStruct((2 * NT * R * CAP,), jnp.int32),
        jax.ShapeDtypeStruct((2 * NT * 16,), jnp.int32),
    ),
    mesh=_mesh,
    scratch_types=[
        pltpu.VMEM((EPT,), jnp.int32),
        pltpu.VMEM((EPT,), jnp.int32),
        pltpu.VMEM((EPT,), jnp.int32),
        pltpu.VMEM((LBUF,), jnp.int32),
        pltpu.VMEM((LBUF,), jnp.int32),
        pltpu.VMEM((LBUF,), jnp.int32),
        pltpu.VMEM((LBUF,), jnp.int32),
        pltpu.VMEM((16,), jnp.int32),
    ],
    compiler_params=pltpu.CompilerParams(needs_layout_passes=False),
)
def _bucket_kernel(src_hbm, dst_hbm, et_hbm, srcl_hbm, dstl_hbm, cnts_hbm,
                   esrc, edst, eet, ls0, ld0, ls1, ld1, cbuf):
    c = lax.axis_index("c")
    s = lax.axis_index("s")
    wid = s * NC + c
    base = wid * EPT
    pltpu.sync_copy(src_hbm.at[pl.ds(base, EPT)], esrc)
    pltpu.sync_copy(dst_hbm.at[pl.ds(base, EPT)], edst)
    pltpu.sync_copy(et_hbm.at[pl.ds(base, EPT)], eet)
    lanes = lax.iota(jnp.int32, 16)
    nullv = jnp.full((16,), NULL, jnp.int32)
    wastev = jnp.full((16,), HALF, jnp.int32)
    tvec0 = jnp.zeros((16,), jnp.int32)
    tvec1 = jnp.zeros((16,), jnp.int32)
    for r in range(R):
        def body(i, carry, r=r):
            cnt0, cnt1 = carry
            off = i * 16
            ev = eet[pl.ds(off, 16)]
            sv = esrc[pl.ds(off, 16)]
            dv = edst[pl.ds(off, 16)]
            m = ev == r
            svp = (sv & 1) * HALF + lax.shift_right_logical(sv, 1)
            dvp = lax.shift_right_logical(dv, 1)
            hi = (dv & 1) == 1
            m0 = m & (~hi)
            m1 = m & hi
            mi0 = m0.astype(jnp.int32)
            incl0 = plsc.cumsum(mi0)
            pos0 = cnt0 + incl0 - mi0
            plsc.store_scatter(ls0, [pos0], svp, mask=m0)
            plsc.store_scatter(ld0, [pos0], dvp, mask=m0)
            mi1 = m1.astype(jnp.int32)
            incl1 = plsc.cumsum(mi1)
            pos1 = cnt1 + incl1 - mi1
            plsc.store_scatter(ls1, [pos1], svp, mask=m1)
            plsc.store_scatter(ld1, [pos1], dvp, mask=m1)
            return (cnt0 + jnp.max(incl0), cnt1 + jnp.max(incl1))
        cnt0, cnt1 = lax.fori_loop(0, NV, body, (jnp.int32(0), jnp.int32(0)))
        for k in range(CHUNK // 16):
            ls0[pl.ds(cnt0 + k * 16, 16)] = nullv
            ld0[pl.ds(cnt0 + k * 16, 16)] = wastev
            ls1[pl.ds(cnt1 + k * 16, 16)] = nullv
            ld1[pl.ds(cnt1 + k * 16, 16)] = wastev
        trips0 = (cnt0 + (CHUNK - 1)) // CHUNK
        trips1 = (cnt1 + (CHUNK - 1)) // CHUNK
        tvec0 = jnp.where(lanes == r, trips0, tvec0)
        tvec1 = jnp.where(lanes == r, trips1, tvec1)
        lb0 = (wid * R + r) * CAP
        lb1 = ((NT + wid) * R + r) * CAP
        pltpu.sync_copy(ls0.at[pl.ds(0, CAP)], srcl_hbm.at[pl.ds(lb0, CAP)])
        pltpu.sync_copy(ld0.at[pl.ds(0, CAP)], dstl_hbm.at[pl.ds(lb0, CAP)])
        pltpu.sync_copy(ls1.at[pl.ds(0, CAP)], srcl_hbm.at[pl.ds(lb1, CAP)])
        pltpu.sync_copy(ld1.at[pl.ds(0, CAP)], dstl_hbm.at[pl.ds(lb1, CAP)])
    cbuf[...] = tvec0
    pltpu.sync_copy(cbuf, cnts_hbm.at[pl.ds(wid * 16, 16)])
    cbuf[...] = tvec1
    pltpu.sync_copy(cbuf, cnts_hbm.at[pl.ds((NT + wid) * 16, 16)])


@functools.partial(
    pl.kernel,
    out_type=jax.ShapeDtypeStruct((R, NPAD, D), jnp.float32),
    mesh=_mesh,
    scratch_types=[
        pltpu.VMEM((CAPB, CHUNK), jnp.int32),
        pltpu.VMEM((CAPB, CHUNK), jnp.int32),
        pltpu.VMEM((CHUNK, D), jnp.float32),
        pltpu.VMEM((CHUNK, D), jnp.float32),
        pltpu.VMEM((CHUNK, D), jnp.float32),
        pltpu.VMEM((16,), jnp.int32),
        pltpu.VMEM((16,), jnp.int32),
        pltpu.SemaphoreType.DMA,
        pltpu.SemaphoreType.DMA,
        pltpu.SemaphoreType.DMA,
        pltpu.VMEM_SHARED((HDIM, D), jnp.float32),
    ],
    compiler_params=pltpu.CompilerParams(needs_layout_passes=False),
)
def _scatter_kernel(x_hbm, srcl_hbm, dstl_hbm, cnts_hbm, zs_hbm, S_hbm,
                    sidx, didx, gb0, gb1, gb2, cbufa, cbufb,
                    sem0, sem1, sem2, S_sh):
    NBUF = 3
    gbufs = (gb0, gb1, gb2)
    sems = (sem0, sem1, sem2)
    c = lax.axis_index("c")
    s = lax.axis_index("s")
    pltpu.sync_copy(cnts_hbm.at[pl.ds((c * NT + s) * 16, 16)], cbufa)
    pltpu.sync_copy(cnts_hbm.at[pl.ds((c * NT + s + NS) * 16, 16)], cbufb)
    lanes = lax.iota(jnp.int32, 16)
    for r in range(R):
        pltpu.sync_copy(zs_hbm, S_sh.at[pl.ds(s * RPT, CHUNK)])
        pltpu.sync_copy(zs_hbm, S_sh.at[pl.ds(s * RPT + CHUNK, CHUNK)])
        pltpu.sync_copy(zs_hbm.at[pl.ds(0, RPT - 2 * CHUNK)],
                        S_sh.at[pl.ds(s * RPT + 2 * CHUNK, RPT - 2 * CHUNK)])
        plsc.subcore_barrier()
        for k in range(2):
            b = s + k * NS
            cvec = (cbufa if k == 0 else cbufb)[...]
            trips = jnp.max(jnp.where(lanes == r, cvec, 0))
            nstage = (trips + 7) // 8

            def stage_body(bk, carry, b=b):
                pltpu.sync_copy(srcl_hbm.at[c, b, r, pl.ds(bk * 8, 8)],
                                sidx.at[pl.ds(bk * 8, 8)])
                pltpu.sync_copy(dstl_hbm.at[c, b, r, pl.ds(bk * 8, 8)],
                                didx.at[pl.ds(bk * 8, 8)])
                return carry
            lax.fori_loop(0, nstage, stage_body, jnp.int32(0))

            for t in range(NBUF):
                @pl.when(t < trips)
                def _(t=t):
                    pltpu.async_copy(x_hbm.at[sidx.at[t]], gbufs[t], sems[t])

            def ring_body(g, carry, trips=trips):
                for t in range(NBUF):
                    j = g * NBUF + t

                    @pl.when(j < trips)
                    def _(j=j, t=t):
                        pltpu.make_async_copy(
                            x_hbm.at[sidx.at[j]], gbufs[t], sems[t]).wait()
                        pltpu.sync_copy(gbufs[t], S_sh.at[didx.at[j]],
                                        add=True)

                    @pl.when(j + NBUF < trips)
                    def _(j=j, t=t):
                        pltpu.async_copy(
                            x_hbm.at[sidx.at[j + NBUF]], gbufs[t], sems[t])
                return carry
            lax.fori_loop(0, (trips + NBUF - 1) // NBUF, ring_body,
                          jnp.int32(0))
        plsc.subcore_barrier()
        pltpu.sync_copy(S_sh.at[pl.ds(s * RPT, RPT)],
                        S_hbm.at[r, pl.ds(c * HALF + s * RPT, RPT)])


@functools.partial(
    pl.kernel,
    out_type=jax.ShapeDtypeStruct((R, NPAD, D), jnp.float32),
    mesh=_mesh,
    scratch_types=[
        pltpu.VMEM((CAPB, CHUNK), jnp.int32),
        pltpu.VMEM((CHUNK, D), jnp.float32),
        pltpu.VMEM((16,), jnp.int32),
        pltpu.VMEM((16,), jnp.int32),
        pltpu.VMEM_SHARED((HDIM, D), jnp.float32),
    ],
    compiler_params=pltpu.CompilerParams(needs_layout_passes=False),
)
def _deg_kernel(dstl_hbm, cnts_hbm, zs_hbm, ones_hbm, deg_hbm,
                didx, ones_b, cbufa, cbufb, deg_sh):
    c = lax.axis_index("c")
    s = lax.axis_index("s")
    pltpu.sync_copy(ones_hbm, ones_b)
    pltpu.sync_copy(cnts_hbm.at[pl.ds((c * NT + s) * 16, 16)], cbufa)
    pltpu.sync_copy(cnts_hbm.at[pl.ds((c * NT + s + NS) * 16, 16)], cbufb)
    lanes = lax.iota(jnp.int32, 16)
    for r in range(R):
        pltpu.sync_copy(zs_hbm, deg_sh.at[pl.ds(s * RPT, CHUNK)])
        pltpu.sync_copy(zs_hbm, deg_sh.at[pl.ds(s * RPT + CHUNK, CHUNK)])
        pltpu.sync_copy(zs_hbm.at[pl.ds(0, RPT - 2 * CHUNK)],
                        deg_sh.at[pl.ds(s * RPT + 2 * CHUNK, RPT - 2 * CHUNK)])
        plsc.subcore_barrier()
        for k in range(2):
            b = s + k * NS
            cvec = (cbufa if k == 0 else cbufb)[...]
            trips = jnp.max(jnp.where(lanes == r, cvec, 0))
            nstage = (trips + 7) // 8

            def stage_body(bk, carry, b=b):
                pltpu.sync_copy(dstl_hbm.at[c, b, r, pl.ds(bk * 8, 8)],
                                didx.at[pl.ds(bk * 8, 8)])
                return carry
            lax.fori_loop(0, nstage, stage_body, jnp.int32(0))

            def chunk_body(j, carry):
                pltpu.sync_copy(ones_b, deg_sh.at[didx.at[j]], add=True)
                return carry
            lax.fori_loop(0, trips, chunk_body, jnp.int32(0))
        plsc.subcore_barrier()
        pltpu.sync_copy(deg_sh.at[pl.ds(s * RPT, RPT)],
                        deg_hbm.at[r, pl.ds(c * HALF + s * RPT, RPT)])


BN = 1024
NBLK = NPAD // BN


def _dense_body(x_ref, S_ref, deg_ref, bases_ref, coeffs_ref, wsl_ref,
                bsl_ref, gamma_ref, beta_ref, out_ref):
    i = pl.program_id(0)
    x = x_ref[...]
    h = jnp.dot(x, wsl_ref[...], preferred_element_type=jnp.float32)
    h = h + bsl_ref[...]
    bases = bases_ref[...]
    cp = coeffs_ref[...]
    for r in range(R):
        W_r = jnp.zeros((D, D), jnp.float32)
        for b in range(NB):
            W_r = W_r + cp[r, b] * bases[b]
        inv = 1.0 / jnp.maximum(deg_ref[r], 1.0)
        h = h + jnp.dot(S_ref[r], W_r, preferred_element_type=jnp.float32) * inv
    h = h * (1.0 / (1.0 + jnp.exp(-h)))
    mu = jnp.mean(h, axis=-1, keepdims=True)
    var = jnp.mean((h - mu) ** 2, axis=-1, keepdims=True)
    h = (h - mu) * lax.rsqrt(var + 1e-5) * gamma_ref[...] + beta_ref[...]
    rowid = i * BN + lax.broadcasted_iota(jnp.int32, (BN, 1), 0)
    rowh = jnp.where(rowid >= HALF, rowid - HALF, rowid)
    out_ref[...] = jnp.where(rowh < N // 2, h, 0.0)


_dense = pl.pallas_call(
    _dense_body,
    grid=(NBLK,),
    in_specs=[
        pl.BlockSpec((BN, D), lambda i: (i, 0)),
        pl.BlockSpec((R, BN, D), lambda i: (0, i, 0)),
        pl.BlockSpec((R, BN, 1), lambda i: (0, i, 0)),
        pl.BlockSpec((NB, D, D), lambda i: (0, 0, 0)),
        pl.BlockSpec((8, 8), lambda i: (0, 0)),
        pl.BlockSpec((D, D), lambda i: (0, 0)),
        pl.BlockSpec((1, D), lambda i: (0, 0)),
        pl.BlockSpec((1, D), lambda i: (0, 0)),
        pl.BlockSpec((1, D), lambda i: (0, 0)),
    ],
    out_specs=pl.BlockSpec((BN, D), lambda i: (i, 0)),
    out_shape=jax.ShapeDtypeStruct((NPAD, D), jnp.float32),
)


def kernel(x, edge_index, edge_type,
           bases_0, coeffs_0, Wsl_0, bsl_0, gamma_0, beta_0,
           bases_1, coeffs_1, Wsl_1, bsl_1, gamma_1, beta_1,
           bases_2, coeffs_2, Wsl_2, bsl_2, gamma_2, beta_2):
    src = edge_index[0].astype(jnp.int32)
    dst = edge_index[1].astype(jnp.int32)
    et = edge_type.astype(jnp.int32)
    x_pad = jnp.zeros((NPAD, D), jnp.float32).at[:N].set(x)
    inv = jnp.concatenate([2 * jnp.arange(HALF, dtype=jnp.int32),
                           2 * jnp.arange(HALF, dtype=jnp.int32) + 1])
    x_perm = jnp.where((inv < N)[:, None], x_pad[jnp.minimum(inv, N)], 0.0)

    srcl, dstl, cnts = _bucket_kernel(src, dst, et)
    srcl5 = srcl.reshape(2, NT, R, CAPB, CHUNK)
    dstl5 = dstl.reshape(2, NT, R, CAPB, CHUNK)

    zs = jnp.zeros((CHUNK, D), jnp.float32)
    ones128 = jnp.ones((CHUNK, D), jnp.float32)

    deg = _deg_kernel(dstl5, cnts, zs, ones128)
    deg1 = deg[:, :, 0:1]

    params = [
        (bases_0, coeffs_0, Wsl_0, bsl_0, gamma_0, beta_0),
        (bases_1, coeffs_1, Wsl_1, bsl_1, gamma_1, beta_1),
        (bases_2, coeffs_2, Wsl_2, bsl_2, gamma_2, beta_2),
    ]
    h = x_perm
    for bases, coeffs, wsl, bsl, gamma, beta in params:
        S = _scatter_kernel(h, srcl5, dstl5, cnts, zs)
        cpad = jnp.zeros((8, 8), jnp.float32).at[:R, :NB].set(coeffs)
        h = _dense(h, S, deg1, bases, cpad, wsl,
                   bsl.reshape(1, D), gamma.reshape(1, D), beta.reshape(1, D))
    v = jnp.arange(N, dtype=jnp.int32)
    return h[(v & 1) * HALF + (v >> 1)]


# speedup vs baseline: 1.6989x; 1.3294x over previous
import functools

import jax
import jax.numpy as jnp
from jax import lax
from jax.experimental import pallas as pl
from jax.experimental.pallas import tpu as pltpu
from jax.experimental.pallas import tpu_sc as plsc

N = 10000
E = 320000
D = 128
R = 7
NB = 4

NC = 2
NS = 16
NT = NC * NS
EPT = E // NT
NV = EPT // 16
CHUNK = 64
CSH = 6
CAPB = -(-EPT // CHUNK)
NPAD = 10240
HALF = NPAD // 2
HDIM = HALF + 8
RPT = HALF // NS
NULL = N // 2

_mesh = plsc.VectorSubcoreMesh(
    core_axis_name="c", subcore_axis_name="s", num_cores=NC, num_subcores=NS)


@functools.partial(
    pl.kernel,
    out_type=(
        jax.ShapeDtypeStruct((2, NT, R, CAPB, CHUNK), jnp.int32),
        jax.ShapeDtypeStruct((2, NT, R, CAPB, CHUNK), jnp.int32),
        jax.ShapeDtypeStruct((2 * NT * 16,), jnp.int32),
    ),
    mesh=_mesh,
    scratch_types=[
        pltpu.VMEM((EPT,), jnp.int32),
        pltpu.VMEM((EPT,), jnp.int32),
        pltpu.VMEM((EPT,), jnp.int32),
        pltpu.VMEM((CAPB + 1, CHUNK), jnp.int32),
        pltpu.VMEM((CAPB + 1, CHUNK), jnp.int32),
        pltpu.VMEM((CAPB + 1, CHUNK), jnp.int32),
        pltpu.VMEM((CAPB + 1, CHUNK), jnp.int32),
        pltpu.VMEM((16,), jnp.int32),
    ],
    compiler_params=pltpu.CompilerParams(needs_layout_passes=False),
)
def _bucket_kernel(src_hbm, dst_hbm, et_hbm, srcl_hbm, dstl_hbm, cnts_hbm,
                   esrc, edst, eet, ls0, ld0, ls1, ld1, cbuf):
    c = lax.axis_index("c")
    s = lax.axis_index("s")
    wid = s * NC + c
    base = wid * EPT
    pltpu.sync_copy(src_hbm.at[pl.ds(base, EPT)], esrc)
    pltpu.sync_copy(dst_hbm.at[pl.ds(base, EPT)], edst)
    pltpu.sync_copy(et_hbm.at[pl.ds(base, EPT)], eet)
    lanes = lax.iota(jnp.int32, 16)
    nullv = jnp.full((16,), NULL, jnp.int32)
    wastev = jnp.full((16,), HALF, jnp.int32)
    tvec0 = jnp.zeros((16,), jnp.int32)
    tvec1 = jnp.zeros((16,), jnp.int32)
    for r in range(R):
        def body(i, carry, r=r):
            cnt0, cnt1 = carry
            off = i * 16
            ev = eet[pl.ds(off, 16)]
            sv = esrc[pl.ds(off, 16)]
            dv = edst[pl.ds(off, 16)]
            m = ev == r
            svp = (sv & 1) * HALF + lax.shift_right_logical(sv, 1)
            dvp = lax.shift_right_logical(dv, 1)
            hi = (dv & 1) == 1
            m0 = m & (~hi)
            m1 = m & hi
            mi0 = m0.astype(jnp.int32)
            incl0 = plsc.cumsum(mi0)
            pos0 = cnt0 + incl0 - mi0
            plsc.store_scatter(ls0, [lax.shift_right_logical(pos0, CSH),
                                     pos0 & (CHUNK - 1)], svp, mask=m0)
            plsc.store_scatter(ld0, [lax.shift_right_logical(pos0, CSH),
                                     pos0 & (CHUNK - 1)], dvp, mask=m0)
            mi1 = m1.astype(jnp.int32)
            incl1 = plsc.cumsum(mi1)
            pos1 = cnt1 + incl1 - mi1
            plsc.store_scatter(ls1, [lax.shift_right_logical(pos1, CSH),
                                     pos1 & (CHUNK - 1)], svp, mask=m1)
            plsc.store_scatter(ld1, [lax.shift_right_logical(pos1, CSH),
                                     pos1 & (CHUNK - 1)], dvp, mask=m1)
            return (cnt0 + jnp.max(incl0), cnt1 + jnp.max(incl1))
        cnt0, cnt1 = lax.fori_loop(0, NV, body, (jnp.int32(0), jnp.int32(0)))
        for k in range(CHUNK // 16):
            p0 = cnt0 + k * 16 + lanes
            plsc.store_scatter(ls0, [lax.shift_right_logical(p0, CSH),
                                     p0 & (CHUNK - 1)], nullv)
            plsc.store_scatter(ld0, [lax.shift_right_logical(p0, CSH),
                                     p0 & (CHUNK - 1)], wastev)
            p1 = cnt1 + k * 16 + lanes
            plsc.store_scatter(ls1, [lax.shift_right_logical(p1, CSH),
                                     p1 & (CHUNK - 1)], nullv)
            plsc.store_scatter(ld1, [lax.shift_right_logical(p1, CSH),
                                     p1 & (CHUNK - 1)], wastev)
        trips0 = (cnt0 + (CHUNK - 1)) // CHUNK
        trips1 = (cnt1 + (CHUNK - 1)) // CHUNK
        tvec0 = jnp.where(lanes == r, trips0, tvec0)
        tvec1 = jnp.where(lanes == r, trips1, tvec1)
        pltpu.sync_copy(ls0.at[pl.ds(0, CAPB)], srcl_hbm.at[0, wid, r])
        pltpu.sync_copy(ld0.at[pl.ds(0, CAPB)], dstl_hbm.at[0, wid, r])
        pltpu.sync_copy(ls1.at[pl.ds(0, CAPB)], srcl_hbm.at[1, wid, r])
        pltpu.sync_copy(ld1.at[pl.ds(0, CAPB)], dstl_hbm.at[1, wid, r])
    cbuf[...] = tvec0
    pltpu.sync_copy(cbuf, cnts_hbm.at[pl.ds(wid * 16, 16)])
    cbuf[...] = tvec1
    pltpu.sync_copy(cbuf, cnts_hbm.at[pl.ds((NT + wid) * 16, 16)])


@functools.partial(
    pl.kernel,
    out_type=jax.ShapeDtypeStruct((R, NPAD, D), jnp.float32),
    mesh=_mesh,
    scratch_types=[
        pltpu.VMEM((CAPB, CHUNK), jnp.int32),
        pltpu.VMEM((CAPB, CHUNK), jnp.int32),
        pltpu.VMEM((CHUNK, D), jnp.float32),
        pltpu.VMEM((CHUNK, D), jnp.float32),
        pltpu.VMEM((CHUNK, D), jnp.float32),
        pltpu.VMEM((16,), jnp.int32),
        pltpu.VMEM((16,), jnp.int32),
        pltpu.SemaphoreType.DMA,
        pltpu.SemaphoreType.DMA,
        pltpu.SemaphoreType.DMA,
        pltpu.VMEM_SHARED((HDIM, D), jnp.float32),
    ],
    compiler_params=pltpu.CompilerParams(needs_layout_passes=False),
)
def _scatter_kernel(x_hbm, srcl_hbm, dstl_hbm, cnts_hbm, zs_hbm, S_hbm,
                    sidx, didx, gb0, gb1, gb2, cbufa, cbufb,
                    sem0, sem1, sem2, S_sh):
    NBUF = 3
    gbufs = (gb0, gb1, gb2)
    sems = (sem0, sem1, sem2)
    c = lax.axis_index("c")
    s = lax.axis_index("s")
    pltpu.sync_copy(cnts_hbm.at[pl.ds((c * NT + s) * 16, 16)], cbufa)
    pltpu.sync_copy(cnts_hbm.at[pl.ds((c * NT + s + NS) * 16, 16)], cbufb)
    lanes = lax.iota(jnp.int32, 16)
    for r in range(R):
        for kk in range(RPT // CHUNK):
            pltpu.sync_copy(zs_hbm,
                            S_sh.at[pl.ds(s * RPT + kk * CHUNK, CHUNK)])
        plsc.subcore_barrier()
        for k in range(2):
            b = s + k * NS
            cvec = (cbufa if k == 0 else cbufb)[...]
            trips = jnp.max(jnp.where(lanes == r, cvec, 0))
            nstage = (trips + 7) // 8

            def stage_body(bk, carry, b=b):
                pltpu.sync_copy(srcl_hbm.at[c, b, r, pl.ds(bk * 8, 8)],
                                sidx.at[pl.ds(bk * 8, 8)])
                pltpu.sync_copy(dstl_hbm.at[c, b, r, pl.ds(bk * 8, 8)],
                                didx.at[pl.ds(bk * 8, 8)])
                return carry
            lax.fori_loop(0, nstage, stage_body, jnp.int32(0))

            for t in range(NBUF):
                @pl.when(t < trips)
                def _(t=t):
                    pltpu.async_copy(x_hbm.at[sidx.at[t]], gbufs[t], sems[t])

            def ring_body(g, carry, trips=trips):
                for t in range(NBUF):
                    j = g * NBUF + t

                    @pl.when(j < trips)
                    def _(j=j, t=t):
                        pltpu.make_async_copy(
                            x_hbm.at[sidx.at[j]], gbufs[t], sems[t]).wait()
                        pltpu.sync_copy(gbufs[t], S_sh.at[didx.at[j]],
                                        add=True)

                    @pl.when(j + NBUF < trips)
                    def _(j=j, t=t):
                        pltpu.async_copy(
                            x_hbm.at[sidx.at[j + NBUF]], gbufs[t], sems[t])
                return carry
            lax.fori_loop(0, (trips + NBUF - 1) // NBUF, ring_body,
                          jnp.int32(0))
        plsc.subcore_barrier()
        pltpu.sync_copy(S_sh.at[pl.ds(s * RPT, RPT)],
                        S_hbm.at[r, pl.ds(c * HALF + s * RPT, RPT)])


@functools.partial(
    pl.kernel,
    out_type=jax.ShapeDtypeStruct((R, NPAD, D), jnp.float32),
    mesh=_mesh,
    scratch_types=[
        pltpu.VMEM((CAPB, CHUNK), jnp.int32),
        pltpu.VMEM((CHUNK, D), jnp.float32),
        pltpu.VMEM((16,), jnp.int32),
        pltpu.VMEM((16,), jnp.int32),
        pltpu.VMEM_SHARED((HDIM, D), jnp.float32),
    ],
    compiler_params=pltpu.CompilerParams(needs_layout_passes=False),
)
def _deg_kernel(dstl_hbm, cnts_hbm, zs_hbm, ones_hbm, deg_hbm,
                didx, ones_b, cbufa, cbufb, deg_sh):
    c = lax.axis_index("c")
    s = lax.axis_index("s")
    pltpu.sync_copy(ones_hbm, ones_b)
    pltpu.sync_copy(cnts_hbm.at[pl.ds((c * NT + s) * 16, 16)], cbufa)
    pltpu.sync_copy(cnts_hbm.at[pl.ds((c * NT + s + NS) * 16, 16)], cbufb)
    lanes = lax.iota(jnp.int32, 16)
    for r in range(R):
        for kk in range(RPT // CHUNK):
            pltpu.sync_copy(zs_hbm,
                            deg_sh.at[pl.ds(s * RPT + kk * CHUNK, CHUNK)])
        plsc.subcore_barrier()
        for k in range(2):
            b = s + k * NS
            cvec = (cbufa if k == 0 else cbufb)[...]
            trips = jnp.max(jnp.where(lanes == r, cvec, 0))
            nstage = (trips + 7) // 8

            def stage_body(bk, carry, b=b):
                pltpu.sync_copy(dstl_hbm.at[c, b, r, pl.ds(bk * 8, 8)],
                                didx.at[pl.ds(bk * 8, 8)])
                return carry
            lax.fori_loop(0, nstage, stage_body, jnp.int32(0))

            def chunk_body(j, carry):
                pltpu.sync_copy(ones_b, deg_sh.at[didx.at[j]], add=True)
                return carry
            lax.fori_loop(0, trips, chunk_body, jnp.int32(0))
        plsc.subcore_barrier()
        pltpu.sync_copy(deg_sh.at[pl.ds(s * RPT, RPT)],
                        deg_hbm.at[r, pl.ds(c * HALF + s * RPT, RPT)])


BN = 1024
NBLK = NPAD // BN


def _dense_body(x_ref, S_ref, deg_ref, bases_ref, coeffs_ref, wsl_ref,
                bsl_ref, gamma_ref, beta_ref, out_ref):
    i = pl.program_id(0)
    x = x_ref[...]
    h = jnp.dot(x, wsl_ref[...], preferred_element_type=jnp.float32)
    h = h + bsl_ref[...]
    bases = bases_ref[...]
    cp = coeffs_ref[...]
    for r in range(R):
        W_r = jnp.zeros((D, D), jnp.float32)
        for b in range(NB):
            W_r = W_r + cp[r, b] * bases[b]
        inv = 1.0 / jnp.maximum(deg_ref[r], 1.0)
        h = h + jnp.dot(S_ref[r], W_r, preferred_element_type=jnp.float32) * inv
    h = h * (1.0 / (1.0 + jnp.exp(-h)))
    mu = jnp.mean(h, axis=-1, keepdims=True)
    var = jnp.mean((h - mu) ** 2, axis=-1, keepdims=True)
    h = (h - mu) * lax.rsqrt(var + 1e-5) * gamma_ref[...] + beta_ref[...]
    rowid = i * BN + lax.broadcasted_iota(jnp.int32, (BN, 1), 0)
    rowh = jnp.where(rowid >= HALF, rowid - HALF, rowid)
    out_ref[...] = jnp.where(rowh < N // 2, h, 0.0)


_dense = pl.pallas_call(
    _dense_body,
    grid=(NBLK,),
    in_specs=[
        pl.BlockSpec((BN, D), lambda i: (i, 0)),
        pl.BlockSpec((R, BN, D), lambda i: (0, i, 0)),
        pl.BlockSpec((R, BN, 1), lambda i: (0, i, 0)),
        pl.BlockSpec((NB, D, D), lambda i: (0, 0, 0)),
        pl.BlockSpec((8, 8), lambda i: (0, 0)),
        pl.BlockSpec((D, D), lambda i: (0, 0)),
        pl.BlockSpec((1, D), lambda i: (0, 0)),
        pl.BlockSpec((1, D), lambda i: (0, 0)),
        pl.BlockSpec((1, D), lambda i: (0, 0)),
    ],
    out_specs=pl.BlockSpec((BN, D), lambda i: (i, 0)),
    out_shape=jax.ShapeDtypeStruct((NPAD, D), jnp.float32),
)


def kernel(x, edge_index, edge_type,
           bases_0, coeffs_0, Wsl_0, bsl_0, gamma_0, beta_0,
           bases_1, coeffs_1, Wsl_1, bsl_1, gamma_1, beta_1,
           bases_2, coeffs_2, Wsl_2, bsl_2, gamma_2, beta_2):
    src = edge_index[0].astype(jnp.int32)
    dst = edge_index[1].astype(jnp.int32)
    et = edge_type.astype(jnp.int32)
    x_pad = jnp.zeros((NPAD, D), jnp.float32).at[:N].set(x)
    inv = jnp.concatenate([2 * jnp.arange(HALF, dtype=jnp.int32),
                           2 * jnp.arange(HALF, dtype=jnp.int32) + 1])
    x_perm = jnp.where((inv < N)[:, None], x_pad[jnp.minimum(inv, N)], 0.0)

    srcl5, dstl5, cnts = _bucket_kernel(src, dst, et)

    zs = jnp.zeros((CHUNK, D), jnp.float32)
    ones128 = jnp.ones((CHUNK, D), jnp.float32)

    deg = _deg_kernel(dstl5, cnts, zs, ones128)
    deg1 = deg[:, :, 0:1]

    params = [
        (bases_0, coeffs_0, Wsl_0, bsl_0, gamma_0, beta_0),
        (bases_1, coeffs_1, Wsl_1, bsl_1, gamma_1, beta_1),
        (bases_2, coeffs_2, Wsl_2, bsl_2, gamma_2, beta_2),
    ]
    h = x_perm
    for bases, coeffs, wsl, bsl, gamma, beta in params:
        S = _scatter_kernel(h, srcl5, dstl5, cnts, zs)
        cpad = jnp.zeros((8, 8), jnp.float32).at[:R, :NB].set(coeffs)
        h = _dense(h, S, deg1, bases, cpad, wsl,
                   bsl.reshape(1, D), gamma.reshape(1, D), beta.reshape(1, D))
    v = jnp.arange(N, dtype=jnp.int32)
    return h[(v & 1) * HALF + (v >> 1)]
